# speedup vs baseline: 3.8843x; 3.8843x over previous
import functools

import jax
import jax.numpy as jnp
from jax.experimental import pallas as pl
from jax.experimental.pallas import tpu as pltpu

N = 20000
NUM_CLASSES = 81
C = NUM_CLASSES - 1
K = 300
KP = 304
CHUNK = 8
N_CHUNKS = KP // CHUNK
NMS_T = 0.3
SCORE_T = 0.05
IMG_W = 800.0
IMG_H = 800.0


def _nms_body(rx1, ry1, rx2, ry2, ddx, ddy, ddw, ddh, sc,
              ox1, oy1, ox2, oy2, osc,
              bx1_s, by1_s, bx2_s, by2_s, area_s, keep_s, iou_s):
    w = rx2[:] - rx1[:] + 1.0
    h = ry2[:] - ry1[:] + 1.0
    cx = rx1[:] + 0.5 * w
    cy = ry1[:] + 0.5 * h
    pcx = (ddx[:] * 0.1) * w + cx
    pcy = (ddy[:] * 0.1) * h + cy
    pw = jnp.exp(ddw[:] * 0.2) * w
    ph = jnp.exp(ddh[:] * 0.2) * h
    x1 = jnp.clip(pcx - 0.5 * pw, 0.0, IMG_W - 1.0)
    y1 = jnp.clip(pcy - 0.5 * ph, 0.0, IMG_H - 1.0)
    x2 = jnp.clip(pcx + 0.5 * pw, 0.0, IMG_W - 1.0)
    y2 = jnp.clip(pcy + 0.5 * ph, 0.0, IMG_H - 1.0)
    bx1_s[:] = x1
    by1_s[:] = y1
    bx2_s[:] = x2
    by2_s[:] = y2
    area_s[:] = jnp.maximum(x2 - x1, 0.0) * jnp.maximum(y2 - y1, 0.0)
    keep_s[:] = (sc[:] > SCORE_T).astype(jnp.float32)

    iota_kp = jax.lax.broadcasted_iota(jnp.int32, (KP, 1), 0)
    iota8 = jax.lax.broadcasted_iota(jnp.int32, (CHUNK, 1), 0)

    def chunk_step(ci, _):
        i0 = ci * CHUNK
        bx1v = bx1_s[:]
        by1v = by1_s[:]
        bx2v = bx2_s[:]
        by2v = by2_s[:]
        areav = area_s[:]
        for t in range(CHUNK):
            cb_x1 = bx1_s[pl.ds(i0 + t, 1), :]
            cb_y1 = by1_s[pl.ds(i0 + t, 1), :]
            cb_x2 = bx2_s[pl.ds(i0 + t, 1), :]
            cb_y2 = by2_s[pl.ds(i0 + t, 1), :]
            cb_a = area_s[pl.ds(i0 + t, 1), :]
            ltx = jnp.maximum(bx1v, cb_x1)
            lty = jnp.maximum(by1v, cb_y1)
            rbx = jnp.minimum(bx2v, cb_x2)
            rby = jnp.minimum(by2v, cb_y2)
            wx = jnp.maximum(rbx - ltx, 0.0)
            wy = jnp.maximum(rby - lty, 0.0)
            inter = wx * wy
            iou = inter / (areav + cb_a - inter + 1e-9)
            iou_s[t] = iou
        for t in range(CHUNK):
            kt = keep_s[pl.ds(i0 + t, 1), :]
            row_cc = iou_s[t, pl.ds(i0, CHUNK), :]
            supp = ((row_cc > NMS_T).astype(jnp.float32) * kt
                    * (iota8 > t).astype(jnp.float32))
            keep_s[pl.ds(i0, CHUNK), :] = (
                keep_s[pl.ds(i0, CHUNK), :] * (1.0 - supp))
        acc = jnp.zeros((KP, C), jnp.float32)
        for t in range(CHUNK):
            kt = keep_s[pl.ds(i0 + t, 1), :]
            acc = jnp.maximum(acc, (iou_s[t] > NMS_T).astype(jnp.float32) * kt)
        after = (iota_kp >= i0 + CHUNK).astype(jnp.float32)
        keep_s[:] = keep_s[:] * (1.0 - acc * after)
        return 0

    jax.lax.fori_loop(0, N_CHUNKS, chunk_step, 0)

    k = keep_s[:]
    ox1[:] = bx1_s[:] * k
    oy1[:] = by1_s[:] * k
    ox2[:] = bx2_s[:] * k
    oy2[:] = by2_s[:] * k
    osc[:] = sc[:] * k


def _nms_call(*ins):
    f32 = jnp.float32
    outs = pl.pallas_call(
        _nms_body,
        out_shape=[jax.ShapeDtypeStruct((KP, C), f32) for _ in range(5)],
        scratch_shapes=[pltpu.VMEM((KP, C), f32) for _ in range(6)]
        + [pltpu.VMEM((CHUNK, KP, C), f32)],
    )(*ins)
    return outs


def kernel(rois, roi_bbox_pred, roi_cls_scores):
    probs = jax.nn.softmax(roi_cls_scores, axis=-1)
    scores_c = jnp.transpose(probs[:, 1:])
    sc, idx = jax.lax.top_k(scores_c, K)
    r = jnp.take(rois, idx, axis=0)
    d = roi_bbox_pred.reshape(N, NUM_CLASSES, 4)
    dsel = d[idx, jnp.arange(1, NUM_CLASSES)[:, None], :]

    def tp(a):
        a = jnp.transpose(a)
        return jnp.pad(a, ((0, KP - K), (0, 0)))

    ins = ([tp(r[..., i]) for i in range(4)]
           + [tp(dsel[..., i]) for i in range(4)]
           + [tp(sc)])
    ox1, oy1, ox2, oy2, osc = _nms_call(*ins)
    out = jnp.stack([ox1, oy1, ox2, oy2, osc], axis=-1)
    return jnp.transpose(out, (1, 0, 2))[:, :K, :]
